# baseline (speedup 1.0000x reference)
import jax
import jax.numpy as jnp
from jax.experimental import pallas as pl
from jax.experimental.pallas import tpu as pltpu

_B, _H, _W, _C = 2, 64, 512, 64
_OH, _OW = 32, 256
_KH, _KW = 6, 12
_K = 16
_R2 = 2.5 * 2.5
_CIN = _C + 3
_PAD_L, _PAD_R = 3, 5
_WP = _OW + _PAD_L + _PAD_R


def _sa_body(ge, go, nx, w1r, b1r, w2r, b2r, out):
    f32 = jnp.float32
    ow = jax.lax.broadcasted_iota(jnp.int32, (_OW, 1), 0)
    w1 = w1r[...]
    b1 = b1r[...]
    w2 = w2r[...]
    b2 = b2r[...]
    w1b = w1.astype(jnp.bfloat16)
    w2b = w2.astype(jnp.bfloat16)
    colok_q = [jnp.logical_and(ow + q >= 0, ow + q <= _OW - 1)
               for q in range(-3, 3)]

    def per_row(oh, carry):
        new_xyz = nx[0, oh]
        ctr = ge[0, 2 * oh, _PAD_L:_PAD_L + _OW, 0:3]
        corr = (new_xyz[:, 0:1] * w1[0:1, :]
                + new_xyz[:, 1:2] * w1[1:2, :]
                + new_xyz[:, 2:3] * w1[2:3, :])
        corr12 = jnp.concatenate([corr] * _KW, axis=0)
        ctr12 = jnp.concatenate([ctr] * _KW, axis=0)
        cnt = jnp.zeros((_OW, 1), f32)
        acc = jnp.zeros((_OW, 128), f32)
        for ih in range(_KH):
            dh = ih - _KH // 2
            row = 2 * oh + dh
            rowok = jnp.logical_and(row >= 0, row < _H)
            rowc = jnp.clip(row, 0, _H - 1)
            views = []
            for iw in range(_KW):
                q = (iw - _KW // 2) // 2
                plane = ge if iw % 2 == 0 else go
                views.append(plane[0, rowc, _PAD_L + q:_PAD_L + q + _OW, :])
            stack = jnp.concatenate(views, axis=0)
            h1 = jnp.dot(stack.astype(jnp.bfloat16), w1b,
                         preferred_element_type=f32)
            h1 = jnp.maximum(h1 + b1 - corr12, 0.0)
            h2 = jnp.dot(h1.astype(jnp.bfloat16), w2b,
                         preferred_element_type=f32)
            h2 = jnp.maximum(h2 + b2, 0.0)
            dxyz = stack[:, 0:3] - ctr12
            d2s = jnp.sum(dxyz * dxyz, axis=1, keepdims=True)
            for iw in range(_KW):
                q = (iw - _KW // 2) // 2
                d2 = d2s[iw * _OW:(iw + 1) * _OW, :]
                valid = jnp.logical_and(jnp.logical_and(colok_q[q + 3], rowok),
                                        d2 < _R2)
                sel = jnp.logical_and(valid, cnt < float(_K))
                cnt = cnt + valid.astype(f32)
                cand = h2[iw * _OW:(iw + 1) * _OW, :]
                acc = jnp.where(sel, jnp.maximum(acc, cand), acc)
        h1p = jnp.maximum(b1 - corr, 0.0)
        h2p = jnp.maximum(jnp.dot(h1p, w2, preferred_element_type=f32) + b2,
                          0.0)
        acc = jnp.where(cnt < float(_K), jnp.maximum(acc, h2p), acc)
        out[0, oh] = acc
        return carry

    jax.lax.fori_loop(0, _OH, per_row, 0)


def kernel(xyz_proj, points_proj, xyz_sampled_proj, W1, b1, W2, b2):
    f32 = jnp.float32
    ge = jnp.concatenate([xyz_proj[:, :, 0::2, :], points_proj[:, :, 0::2, :]],
                         axis=-1)
    go = jnp.concatenate([xyz_proj[:, :, 1::2, :], points_proj[:, :, 1::2, :]],
                         axis=-1)
    ge = jnp.pad(ge, ((0, 0), (0, 0), (_PAD_L, _PAD_R), (0, 0)))
    go = jnp.pad(go, ((0, 0), (0, 0), (_PAD_L, _PAD_R), (0, 0)))
    b1r = b1.reshape(1, 64)
    b2r = b2.reshape(1, 128)
    grid = (_B,)
    proj = pl.pallas_call(
        _sa_body,
        grid=grid,
        in_specs=[
            pl.BlockSpec((1, _H, _WP, _CIN), lambda b: (b, 0, 0, 0)),
            pl.BlockSpec((1, _H, _WP, _CIN), lambda b: (b, 0, 0, 0)),
            pl.BlockSpec((1, _OH, _OW, 3), lambda b: (b, 0, 0, 0)),
            pl.BlockSpec((_CIN, 64), lambda b: (0, 0)),
            pl.BlockSpec((1, 64), lambda b: (0, 0)),
            pl.BlockSpec((64, 128), lambda b: (0, 0)),
            pl.BlockSpec((1, 128), lambda b: (0, 0)),
        ],
        out_specs=pl.BlockSpec((1, _OH, _OW, 128), lambda b: (b, 0, 0, 0)),
        out_shape=jax.ShapeDtypeStruct((_B, _OH, _OW, 128), f32),
        compiler_params=pltpu.CompilerParams(
            dimension_semantics=("parallel",)),
    )(ge, go, xyz_sampled_proj, W1, b1r, W2, b2r)
    pds = proj.reshape(_B, _OH * _OW, 128)
    return (pds, proj)


# speedup vs baseline: 2.0543x; 2.0543x over previous
import jax
import jax.numpy as jnp
from jax.experimental import pallas as pl
from jax.experimental.pallas import tpu as pltpu

_B, _H, _W, _C = 2, 64, 512, 64
_OH, _OW = 32, 256
_KH, _KW = 6, 12
_K = 16
_R2 = 2.5 * 2.5
_CIN = _C + 3
_PAD_L, _PAD_R = 3, 5
_WP = _OW + _PAD_L + _PAD_R


def _sa_body(gp, nx, w1r, b1r, w2r, b2r, out):
    f32 = jnp.float32
    ow = jax.lax.broadcasted_iota(jnp.int32, (_OW, 1), 0)
    w1 = w1r[...]
    b1 = b1r[...]
    w2 = w2r[...]
    b2 = b2r[...]
    w1b = w1.astype(jnp.bfloat16)
    w2b = w2.astype(jnp.bfloat16)
    colok_q = [jnp.logical_and(ow + q >= 0, ow + q <= _OW - 1)
               for q in range(-3, 3)]

    def per_row(oh, carry):
        new_xyz = nx[0, oh]
        ctr = gp[0, 2 * oh, 0, _PAD_L:_PAD_L + _OW, 0:3]
        corr = (new_xyz[:, 0:1] * w1[0:1, :]
                + new_xyz[:, 1:2] * w1[1:2, :]
                + new_xyz[:, 2:3] * w1[2:3, :])
        corr12 = jnp.concatenate([corr] * _KW, axis=0)
        ctr12 = jnp.concatenate([ctr] * _KW, axis=0)
        cnt = jnp.zeros((_OW, 1), f32)
        acc = jnp.zeros((_OW, 128), f32)
        for ih in range(_KH):
            dh = ih - _KH // 2
            row = 2 * oh + dh
            rowok = jnp.logical_and(row >= 0, row < _H)
            rowc = jnp.clip(row, 0, _H - 1)
            views = []
            for iw in range(_KW):
                q = (iw - _KW // 2) // 2
                views.append(
                    gp[0, rowc, iw % 2, _PAD_L + q:_PAD_L + q + _OW, :])
            stack = jnp.concatenate(views, axis=0)
            h1 = jnp.dot(stack.astype(jnp.bfloat16), w1b,
                         preferred_element_type=f32)
            h1 = jnp.maximum(h1 + b1 - corr12, 0.0)
            h2 = jnp.dot(h1.astype(jnp.bfloat16), w2b,
                         preferred_element_type=f32)
            h2 = jnp.maximum(h2 + b2, 0.0)
            dxyz = stack[:, 0:3] - ctr12
            d2s = jnp.sum(dxyz * dxyz, axis=1, keepdims=True)
            for iw in range(_KW):
                q = (iw - _KW // 2) // 2
                d2 = d2s[iw * _OW:(iw + 1) * _OW, :]
                valid = jnp.logical_and(jnp.logical_and(colok_q[q + 3], rowok),
                                        d2 < _R2)
                sel = jnp.logical_and(valid, cnt < float(_K))
                cnt = cnt + valid.astype(f32)
                cand = h2[iw * _OW:(iw + 1) * _OW, :]
                acc = jnp.where(sel, jnp.maximum(acc, cand), acc)
        h1p = jnp.maximum(b1 - corr, 0.0)
        h2p = jnp.maximum(jnp.dot(h1p, w2, preferred_element_type=f32) + b2,
                          0.0)
        acc = jnp.where(cnt < float(_K), jnp.maximum(acc, h2p), acc)
        out[0, oh] = acc
        return carry

    jax.lax.fori_loop(0, _OH, per_row, 0)


def kernel(xyz_proj, points_proj, xyz_sampled_proj, W1, b1, W2, b2):
    f32 = jnp.float32
    g = jnp.concatenate([xyz_proj, points_proj], axis=-1)
    g = g.reshape(_B, _H, _OW, 2, _CIN).transpose(0, 1, 3, 2, 4)
    gp = jnp.pad(g, ((0, 0), (0, 0), (0, 0), (_PAD_L, _PAD_R), (0, 0)))
    b1r = b1.reshape(1, 64)
    b2r = b2.reshape(1, 128)
    grid = (_B,)
    proj = pl.pallas_call(
        _sa_body,
        grid=grid,
        in_specs=[
            pl.BlockSpec((1, _H, 2, _WP, _CIN), lambda b: (b, 0, 0, 0, 0)),
            pl.BlockSpec((1, _OH, _OW, 3), lambda b: (b, 0, 0, 0)),
            pl.BlockSpec((_CIN, 64), lambda b: (0, 0)),
            pl.BlockSpec((1, 64), lambda b: (0, 0)),
            pl.BlockSpec((64, 128), lambda b: (0, 0)),
            pl.BlockSpec((1, 128), lambda b: (0, 0)),
        ],
        out_specs=pl.BlockSpec((1, _OH, _OW, 128), lambda b: (b, 0, 0, 0)),
        out_shape=jax.ShapeDtypeStruct((_B, _OH, _OW, 128), f32),
        compiler_params=pltpu.CompilerParams(
            dimension_semantics=("parallel",)),
    )(gp, xyz_sampled_proj, W1, b1r, W2, b2r)
    pds = proj.reshape(_B, _OH * _OW, 128)
    return (pds, proj)
